# baseline (speedup 1.0000x reference)
import functools

import numpy as np
import jax
import jax.numpy as jnp
from jax import lax
from jax.experimental import pallas as pl
from jax.experimental.pallas import tpu as pltpu
from jax.experimental.pallas import tpu_sc as plsc

_VOCAB = 100000
_D = 768
_MAX_LEN = 2048
_LANES = 16


def _pe_table(max_len: int, d_model: int) -> np.ndarray:
    pos = np.arange(max_len, dtype=np.float64)[:, None]
    i = np.arange(0, d_model, 2, dtype=np.float64)
    angle = pos / np.power(10000.0, i / d_model)
    pe = np.zeros((max_len, d_model), dtype=np.float32)
    pe[:, 0::2] = np.sin(angle).astype(np.float32)
    pe[:, 1::2] = np.cos(angle).astype(np.float32)
    return pe


_PE = _pe_table(_MAX_LEN, _D)

_CHUNK = 16
_NBUF = 6


@functools.cache
def _build(batch: int, seq: int, d: int):
    info = plsc.get_sparse_core_info()
    nc, ns = info.num_cores, info.num_subcores
    nw = nc * ns
    total = batch * seq
    ppw = seq // nw
    bpw = total // nw
    chunk = _CHUNK
    nch = bpw // chunk
    cpb = ppw // chunk
    assert seq % nw == 0 and ppw % chunk == 0

    mesh = plsc.VectorSubcoreMesh(core_axis_name="c", subcore_axis_name="s")

    @functools.partial(
        pl.kernel,
        mesh=mesh,
        out_type=jax.ShapeDtypeStruct((batch, seq, d), jnp.float32),
        scratch_types=[
            pltpu.VMEM((bpw,), jnp.int32),
            pltpu.VMEM((_NBUF, chunk, d), jnp.float32),
            pltpu.VMEM((ppw, d), jnp.float32),
        ] + [pltpu.SemaphoreType.DMA] * (2 * _NBUF + 1),
    )
    def emb_kernel(x_hbm, table_hbm, pe_hbm, out_hbm,
                   idx_v, buf_v, pe_v, *sems):
        gsem, wsem, psem = sems[:_NBUF], sems[_NBUF:2 * _NBUF], sems[2 * _NBUF]
        wid = lax.axis_index("s") * nc + lax.axis_index("c")
        pos_base = wid * ppw
        pe_cp = pltpu.async_copy(pe_hbm.at[pl.ds(pos_base, ppw)], pe_v, psem)
        for b in range(batch):
            pltpu.sync_copy(
                x_hbm.at[b, pl.ds(pos_base, ppw)],
                idx_v.at[pl.ds(b * ppw, ppw)])
        pe_cp.wait()

        g_cp = [None] * _NBUF
        wb_cp = [None] * _NBUF
        for t in range(nch + 3):
            if t < nch:
                s = t % _NBUF
                if t >= _NBUF:
                    wb_cp[s].wait()
                g_cp[s] = pltpu.async_copy(
                    table_hbm.at[idx_v.at[pl.ds(t * chunk, chunk)]],
                    buf_v.at[s], gsem[s])
            if 3 <= t:
                q = t - 3
                s = q % _NBUF
                pe_off = (q % cpb) * chunk
                b = q // cpb
                g_cp[s].wait()

                @plsc.parallel_loop(0, chunk, 1, unroll=1)
                def row_add(r, s=s, pe_off=pe_off):
                    for j in range(d // _LANES):
                        sl = pl.ds(j * _LANES, _LANES)
                        plsc.addupdate(buf_v.at[s, r, sl], pe_v[pe_off + r, sl])
                wb_cp[s] = pltpu.async_copy(
                    buf_v.at[s],
                    out_hbm.at[b, pl.ds(pos_base + pe_off, chunk)],
                    wsem[s])
        for t in range(max(0, nch - _NBUF), nch):
            wb_cp[t % _NBUF].wait()

    return emb_kernel


def kernel(x, table):
    batch, seq = x.shape
    d = table.shape[1]
    pe = jnp.asarray(_PE[:seq])
    return _build(batch, seq, d)(x.astype(jnp.int32), table, pe)


# speedup vs baseline: 1.1246x; 1.1246x over previous
import functools

import numpy as np
import jax
import jax.numpy as jnp
from jax import lax
from jax.experimental import pallas as pl
from jax.experimental.pallas import tpu as pltpu
from jax.experimental.pallas import tpu_sc as plsc

_VOCAB = 100000
_D = 768
_MAX_LEN = 2048
_LANES = 16


def _pe_table(max_len: int, d_model: int) -> np.ndarray:
    pos = np.arange(max_len, dtype=np.float64)[:, None]
    i = np.arange(0, d_model, 2, dtype=np.float64)
    angle = pos / np.power(10000.0, i / d_model)
    pe = np.zeros((max_len, d_model), dtype=np.float32)
    pe[:, 0::2] = np.sin(angle).astype(np.float32)
    pe[:, 1::2] = np.cos(angle).astype(np.float32)
    return pe


_PE = _pe_table(_MAX_LEN, _D)

_CHUNK = 16
_NBUF = 4
_GAP = 2


@functools.cache
def _build(batch: int, seq: int, d: int):
    info = plsc.get_sparse_core_info()
    nc, ns = info.num_cores, info.num_subcores
    nw = nc * ns
    total = batch * seq
    ppw = seq // nw
    bpw = total // nw
    chunk = _CHUNK
    nch = bpw // chunk
    cpb = ppw // chunk
    ngrp = nch // _NBUF
    assert seq % nw == 0 and ppw % chunk == 0
    assert cpb == _NBUF and nch % _NBUF == 0 and ngrp >= 2

    mesh = plsc.VectorSubcoreMesh(core_axis_name="c", subcore_axis_name="s")

    @functools.partial(
        pl.kernel,
        mesh=mesh,
        out_type=jax.ShapeDtypeStruct((batch, seq, d), jnp.float32),
        scratch_types=[
            pltpu.VMEM((bpw,), jnp.int32),
            pltpu.VMEM((_NBUF, chunk, d), jnp.float32),
            pltpu.VMEM((ppw, d), jnp.float32),
        ] + [pltpu.SemaphoreType.DMA] * (2 * _NBUF + 1),
    )
    def emb_kernel(x_hbm, table_hbm, pe_hbm, out_hbm,
                   idx_v, buf_v, pe_v, *sems):
        gsem, wsem, psem = sems[:_NBUF], sems[_NBUF:2 * _NBUF], sems[2 * _NBUF]
        wid = lax.axis_index("s") * nc + lax.axis_index("c")
        pos_base = wid * ppw
        pe_cp = pltpu.async_copy(pe_hbm.at[pl.ds(pos_base, ppw)], pe_v, psem)
        for b in range(batch):
            pltpu.sync_copy(
                x_hbm.at[b, pl.ds(pos_base, ppw)],
                idx_v.at[pl.ds(b * ppw, ppw)])
        pe_cp.wait()

        def gather(c, slot):
            pltpu.async_copy(
                table_hbm.at[idx_v.at[pl.ds(c * chunk, chunk)]],
                buf_v.at[slot], gsem[slot])

        def wb_wait(slot):
            pltpu.make_async_copy(
                buf_v.at[slot],
                out_hbm.at[0, pl.ds(pos_base, chunk)],
                wsem[slot]).wait()

        for c in range(_GAP):
            gather(c, c)

        def group(g, _):
            for i in range(_NBUF):
                v = g * _NBUF + i
                sa = (i + _GAP) % _NBUF
                if i < _GAP:
                    @pl.when(g >= 1)
                    def _():
                        wb_wait(sa)
                    gather(v + _GAP, sa)
                else:
                    @pl.when(g < ngrp - 1)
                    def _():
                        wb_wait(sa)
                        gather(v + _GAP, sa)
                pltpu.make_async_copy(
                    table_hbm.at[idx_v.at[pl.ds(0, chunk)]],
                    buf_v.at[i], gsem[i]).wait()
                pe_off = i * chunk

                @plsc.parallel_loop(0, chunk, 1, unroll=1)
                def row_add(r, i=i, pe_off=pe_off):
                    for j in range(d // _LANES):
                        sl = pl.ds(j * _LANES, _LANES)
                        plsc.addupdate(buf_v.at[i, r, sl], pe_v[pe_off + r, sl])

                pltpu.async_copy(
                    buf_v.at[i],
                    out_hbm.at[g, pl.ds(pos_base + pe_off, chunk)],
                    wsem[i])
            return 0

        lax.fori_loop(0, ngrp, group, 0)
        for i in range(_NBUF):
            wb_wait(i)

    return emb_kernel


def kernel(x, table):
    batch, seq = x.shape
    d = table.shape[1]
    pe = jnp.asarray(_PE[:seq])
    return _build(batch, seq, d)(x.astype(jnp.int32), table, pe)
